# baseline (speedup 1.0000x reference)
import functools

import jax
import jax.numpy as jnp
from jax import lax
from jax.experimental import pallas as pl
from jax.experimental.pallas import tpu as pltpu
from jax.experimental.pallas import tpu_sc as plsc

D = 64
T = 4
E = 64
LANES = 16
W = 32
PLANE = 14 * E


def _sc_kernel_body(nc, ids_packed,
                    token_table, feature_table, type_table, seq_table,
                    pos_table, out_hbm,
                    idx0, idx1, w0, w1, rows0, rows1, ps0, ps1, outb0, outb1,
                    sem_idx, sem_g0, sem_g1):
    idxb = [idx0, idx1]
    wb = [w0, w1]
    rowsb = [rows0, rows1]
    psb = [ps0, ps1]
    outb = [outb0, outb1]
    sem_g = [sem_g0, sem_g1]

    wid = lax.axis_index("s") * 2 + lax.axis_index("c")
    g0 = wid * nc

    def gathers(p):
        cps = []
        for k, table in enumerate((token_table, feature_table, type_table)):
            for h in range(T * E // 128):
                cps.append(pltpu.make_async_copy(
                    table.at[idxb[p].at[pl.ds(k * T * E + h * 128, 128)]],
                    rowsb[p].at[k, pl.ds(h * 128, 128)], sem_g[p]))
        cps.append(pltpu.make_async_copy(
            pos_table.at[idxb[p].at[pl.ds(3 * T * E, E)]],
            psb[p].at[0], sem_g[p]))
        cps.append(pltpu.make_async_copy(
            seq_table.at[idxb[p].at[pl.ds(3 * T * E + E, E)]],
            psb[p].at[1], sem_g[p]))
        return cps

    def weights(p):
        base4 = lax.iota(jnp.int32, LANES) * T

        def grp(i, _):
            iv = base4 + i * (LANES * T)
            cnt = jnp.zeros((LANES,), jnp.float32)
            ms = []
            for t in range(T):
                ids = plsc.load_gather(idxb[p], [iv + t])
                m = jnp.where(ids != 0, 1.0, 0.0).astype(jnp.float32)
                ms.append(m)
                cnt = cnt + m
            rv = 1.0 / jnp.maximum(cnt, 1.0)
            for t in range(T):
                plsc.store_scatter(wb[p], [iv + t], ms[t] * rv)
            return 0
        lax.fori_loop(0, E // LANES, grp, 0)

    def compute_and_store(p, c):
        def event_body(e, _):
            e4 = e * T
            wv = wb[p][pl.ds(e4, LANES)]
            for dc in range(D // LANES):
                sl = pl.ds(dc * LANES, LANES)
                acc = psb[p][0, e, sl] + psb[p][1, e, sl]
                for t in range(T):
                    acc = acc + (rowsb[p][0, e4 + t, sl]
                                 + rowsb[p][1, e4 + t, sl]
                                 + rowsb[p][2, e4 + t, sl]) * wv[t]
                outb[p][e, sl] = acc
            return 0
        lax.fori_loop(0, E, event_body, 0)
        pltpu.sync_copy(outb[p], out_hbm.at[pl.ds((g0 + c) * E, E), :])

    pltpu.sync_copy(ids_packed.at[g0], idx0)
    weights(0)
    for cp in gathers(0):
        cp.start()
    cp1 = pltpu.make_async_copy(ids_packed.at[g0 + 1], idx1, sem_idx)
    cp1.start()

    def step(i, _):
        for p in (0, 1):
            c = i * 2 + p
            q = 1 - p

            @pl.when(c + 1 < nc)
            def _prefetch():
                pltpu.make_async_copy(
                    ids_packed.at[g0 + c + 1], idxb[q], sem_idx).wait()
                weights(q)
                for cp in gathers(q):
                    cp.start()

            for cp in gathers(p):
                cp.wait()

            @pl.when(c + 2 < nc)
            def _next_idx():
                pltpu.make_async_copy(
                    ids_packed.at[g0 + c + 2], idxb[p], sem_idx).start()

            compute_and_store(p, c)
        return 0

    lax.fori_loop(0, nc // 2, step, 0)


@functools.partial(jax.jit, static_argnums=())
def kernel(token_ids, feature_ids, type_ids, pos_ids, seq_name_ids,
           token_table, feature_table, type_table, seq_table, pos_table):
    B, S, L, Tt = token_ids.shape
    assert Tt == T and token_table.shape[1] == D
    N = B * S * L
    assert N % (W * E) == 0
    nc = N // (W * E)
    assert nc % 2 == 0

    ids_packed = jnp.concatenate([
        token_ids.reshape(N // E, T * E).astype(jnp.int32),
        feature_ids.reshape(N // E, T * E).astype(jnp.int32),
        type_ids.reshape(N // E, T * E).astype(jnp.int32),
        pos_ids.reshape(N // E, E).astype(jnp.int32),
        seq_name_ids.reshape(N // E, E).astype(jnp.int32),
    ], axis=1)

    mesh = plsc.VectorSubcoreMesh(core_axis_name="c", subcore_axis_name="s",
                                  num_cores=2, num_subcores=16)
    kfn = pl.kernel(
        functools.partial(_sc_kernel_body, nc),
        out_type=jax.ShapeDtypeStruct((N, D), jnp.float32),
        mesh=mesh,
        compiler_params=pltpu.CompilerParams(use_tc_tiling_on_sc=False,
                                             needs_layout_passes=False),
        scratch_types=[
            pltpu.VMEM((PLANE,), jnp.int32),
            pltpu.VMEM((PLANE,), jnp.int32),
            pltpu.VMEM((T * E + LANES,), jnp.float32),
            pltpu.VMEM((T * E + LANES,), jnp.float32),
            pltpu.VMEM((3, T * E, D), jnp.float32),
            pltpu.VMEM((3, T * E, D), jnp.float32),
            pltpu.VMEM((2, E, D), jnp.float32),
            pltpu.VMEM((2, E, D), jnp.float32),
            pltpu.VMEM((E, D), jnp.float32),
            pltpu.VMEM((E, D), jnp.float32),
            pltpu.SemaphoreType.DMA,
            pltpu.SemaphoreType.DMA,
            pltpu.SemaphoreType.DMA,
        ],
    )
    out = kfn(ids_packed, token_table, feature_table, type_table,
              seq_table, pos_table)
    return out.reshape(B, S, L, D)


# speedup vs baseline: 1.0320x; 1.0320x over previous
import functools

import jax
import jax.numpy as jnp
from jax import lax
from jax.experimental import pallas as pl
from jax.experimental.pallas import tpu as pltpu
from jax.experimental.pallas import tpu_sc as plsc

D = 64
T = 4
E = 128
LANES = 16
W = 32
PLANE = 14 * E
HIMASK = -65536


def _sc_kernel_body(nc, nb, ids_packed,
                    token_table, feature_table, type_table, seq_table,
                    pos_table, out_hbm,
                    idx0, idx1, w0, w1, rows0, rows1, ps0, ps1, outt,
                    sem_idx, sem_g0, sem_g1):
    idxb = [idx0, idx1]
    wb = [w0, w1]
    rowsb = [rows0, rows1]
    psb = [ps0, ps1]
    sem_g = [sem_g0, sem_g1]

    assert nb & (nb - 1) == 0
    shift_nb = nb.bit_length() - 1
    wid = lax.axis_index("s") * 2 + lax.axis_index("c")
    g0 = wid * nc

    def zero_rows(p):
        z32 = jnp.zeros((2 * LANES,), jnp.bfloat16)

        def zr(e, _):
            for t in range(T):
                rowsb[p][t, e, pl.ds(0, 2 * LANES)] = z32
                rowsb[p][t, e, pl.ds(2 * LANES, 2 * LANES)] = z32
            psb[p][e, pl.ds(0, 2 * LANES)] = z32
            psb[p][e, pl.ds(2 * LANES, 2 * LANES)] = z32
            return 0
        lax.fori_loop(0, E, zr, 0)

    def fire_gathers(p):
        for k, table in enumerate((token_table, feature_table, type_table)):
            for t in range(T):
                pltpu.async_copy(
                    table.at[idxb[p].at[pl.ds((k * T + t) * E, E)]],
                    rowsb[p].at[t], sem_g[p], add=True)
        pltpu.async_copy(
            pos_table.at[idxb[p].at[pl.ds(3 * T * E, E)]],
            psb[p], sem_g[p], add=True)
        pltpu.async_copy(
            seq_table.at[idxb[p].at[pl.ds(3 * T * E + E, E)]],
            psb[p], sem_g[p], add=True)

    def wait_gathers(p):
        for k, table in enumerate((token_table, feature_table, type_table)):
            for t in range(T):
                pltpu.make_async_copy(
                    table.at[idxb[p].at[pl.ds((k * T + t) * E, E)]],
                    rowsb[p].at[t], sem_g[p]).wait()
        pltpu.make_async_copy(
            pos_table.at[idxb[p].at[pl.ds(3 * T * E, E)]],
            psb[p], sem_g[p]).wait()
        pltpu.make_async_copy(
            seq_table.at[idxb[p].at[pl.ds(3 * T * E + E, E)]],
            psb[p], sem_g[p]).wait()

    def weights(p):
        ilane = lax.iota(jnp.int32, LANES)

        def grp(i, _):
            cnt = jnp.zeros((LANES,), jnp.float32)
            ms = []
            for t in range(T):
                ids = idxb[p][pl.ds(t * E + i * LANES, LANES)]
                m = jnp.where(ids != 0, 1.0, 0.0).astype(jnp.float32)
                ms.append(m)
                cnt = cnt + m
            rv = 1.0 / jnp.maximum(cnt, 1.0)
            widx = (ilane + i * LANES) * T
            for t in range(T):
                plsc.store_scatter(wb[p], [widx + t], ms[t] * rv)
            return 0
        lax.fori_loop(0, E // LANES, grp, 0)

    def compute_and_store(p, c):
        iota2 = lax.iota(jnp.int32, LANES) * 2

        def widen(ref, j, e, dc):
            v = plsc.bitcast(
                ref[j, e, pl.ds(dc * 2 * LANES, 2 * LANES)] if j is not None
                else ref[e, pl.ds(dc * 2 * LANES, 2 * LANES)], jnp.int32)
            even = plsc.bitcast(lax.shift_left(v, 16), jnp.float32)
            odd = plsc.bitcast(lax.bitwise_and(v, jnp.int32(HIMASK)),
                               jnp.float32)
            return even, odd

        def event_body(e, _):
            wv = wb[p][pl.ds(e * T, LANES)]
            ev = jnp.zeros((LANES,), jnp.int32) + e
            for dc in range(D // (2 * LANES)):
                acc_e, acc_o = widen(psb[p], None, e, dc)
                for t in range(T):
                    re, ro = widen(rowsb[p], t, e, dc)
                    acc_e = acc_e + re * wv[t]
                    acc_o = acc_o + ro * wv[t]
                plsc.store_scatter(outt, [iota2 + dc * 2 * LANES, ev], acc_e)
                plsc.store_scatter(outt, [iota2 + (dc * 2 * LANES + 1), ev],
                                   acc_o)
            return 0
        lax.fori_loop(0, E, event_body, 0)

        g = g0 + c
        sl_ix = lax.shift_right_logical(g, shift_nb)
        bb = lax.bitwise_and(g, nb - 1)
        pltpu.sync_copy(
            outt, out_hbm.at[sl_ix, :, pl.ds(bb * E, E)])

    pltpu.sync_copy(ids_packed.at[g0], idx0)
    weights(0)
    zero_rows(0)
    fire_gathers(0)
    pltpu.make_async_copy(ids_packed.at[g0 + 1], idx1, sem_idx).start()

    def step(i, _):
        for p in (0, 1):
            c = i * 2 + p
            q = 1 - p

            @pl.when(c + 1 < nc)
            def _prefetch():
                pltpu.make_async_copy(
                    ids_packed.at[g0 + c + 1], idxb[q], sem_idx).wait()
                weights(q)
                zero_rows(q)
                fire_gathers(q)

            wait_gathers(p)

            @pl.when(c + 2 < nc)
            def _next_idx():
                pltpu.make_async_copy(
                    ids_packed.at[g0 + c + 2], idxb[p], sem_idx).start()

            compute_and_store(p, c)
        return 0

    lax.fori_loop(0, nc // 2, step, 0)


@functools.partial(jax.jit, static_argnums=())
def kernel(token_ids, feature_ids, type_ids, pos_ids, seq_name_ids,
           token_table, feature_table, type_table, seq_table, pos_table):
    B, S, L, Tt = token_ids.shape
    assert Tt == T and token_table.shape[1] == D
    SL = S * L
    N = B * SL
    nb = B // E
    NCHUNKS = SL * nb
    assert NCHUNKS % W == 0
    nc = NCHUNKS // W
    assert nc % 2 == 0

    def plane3(x):
        return (x.astype(jnp.int32).transpose(1, 2, 3, 0)
                .reshape(SL, T, nb, E).transpose(0, 2, 1, 3))

    def plane1(x):
        return (x.astype(jnp.int32).transpose(1, 2, 0)
                .reshape(SL, nb, 1, E))

    ids_packed = jnp.concatenate(
        [plane3(token_ids), plane3(feature_ids), plane3(type_ids),
         plane1(pos_ids), plane1(seq_name_ids)],
        axis=2).reshape(NCHUNKS, PLANE)

    bf = jnp.bfloat16
    mesh = plsc.VectorSubcoreMesh(core_axis_name="c", subcore_axis_name="s",
                                  num_cores=2, num_subcores=16)
    kfn = pl.kernel(
        functools.partial(_sc_kernel_body, nc, nb),
        out_type=jax.ShapeDtypeStruct((SL, D, B), jnp.float32),
        mesh=mesh,
        compiler_params=pltpu.CompilerParams(use_tc_tiling_on_sc=False,
                                             needs_layout_passes=False),
        scratch_types=[
            pltpu.VMEM((PLANE,), jnp.int32),
            pltpu.VMEM((PLANE,), jnp.int32),
            pltpu.VMEM((T * E + LANES,), jnp.float32),
            pltpu.VMEM((T * E + LANES,), jnp.float32),
            pltpu.VMEM((T, E, D), bf),
            pltpu.VMEM((T, E, D), bf),
            pltpu.VMEM((E, D), bf),
            pltpu.VMEM((E, D), bf),
            pltpu.VMEM((D, E), jnp.float32),
            pltpu.SemaphoreType.DMA,
            pltpu.SemaphoreType.DMA,
            pltpu.SemaphoreType.DMA,
        ],
    )
    out = kfn(ids_packed, token_table.astype(bf), feature_table.astype(bf),
              type_table.astype(bf), seq_table.astype(bf),
              pos_table.astype(bf))
    return out.reshape(S, L, D, B).transpose(3, 0, 1, 2)


# speedup vs baseline: 1.1441x; 1.1087x over previous
import functools

import jax
import jax.numpy as jnp
from jax import lax
from jax.experimental import pallas as pl
from jax.experimental.pallas import tpu as pltpu
from jax.experimental.pallas import tpu_sc as plsc

D = 64
T = 4
E = 128
LANES = 16
W = 32
PLANE = 14 * E


def _common(nc, nb):
    assert nb & (nb - 1) == 0
    shift_nb = nb.bit_length() - 1
    wid = lax.axis_index("s") * 2 + lax.axis_index("c")
    g0 = wid * nc
    return shift_nb, g0


def _weights(idxp, wp):
    ilane = lax.iota(jnp.int32, LANES)

    def grp(i, _):
        cnt = jnp.zeros((LANES,), jnp.float32)
        ms = []
        for t in range(T):
            ids = idxp[pl.ds(t * E + i * LANES, LANES)]
            m = jnp.where(ids != 0, 1.0, 0.0).astype(jnp.float32)
            ms.append(m)
            cnt = cnt + m
        rv = 1.0 / jnp.maximum(cnt, 1.0)
        widx = (ilane + i * LANES) * T
        for t in range(T):
            plsc.store_scatter(wp, [widx + t], ms[t] * rv)
        return 0
    lax.fori_loop(0, E // LANES, grp, 0)


def _k1_body(nc, nb, ids_packed,
             feature_table, type_table, seq_table, pos_table, out_hbm,
             idx0, idx1, w0, w1, rows0, rows1, ps0, ps1, outt,
             sem_idx, sem_g0, sem_g1):
    idxb, wb = [idx0, idx1], [w0, w1]
    rowsb, psb = [rows0, rows1], [ps0, ps1]
    sem_g = [sem_g0, sem_g1]
    shift_nb, g0 = _common(nc, nb)

    def zero_rows(p):
        z16 = jnp.zeros((LANES,), jnp.float32)

        def zr(e, _):
            for dc in range(D // LANES):
                sl = pl.ds(dc * LANES, LANES)
                for t in range(T):
                    rowsb[p][t, e, sl] = z16
                psb[p][e, sl] = z16
            return 0
        lax.fori_loop(0, E, zr, 0)

    def g_list(p):
        descs = []
        for k, table in enumerate((feature_table, type_table)):
            for t in range(T):
                descs.append(
                    (table.at[idxb[p].at[pl.ds(((k + 1) * T + t) * E, E)]],
                     rowsb[p].at[t], True))
        descs.append((pos_table.at[idxb[p].at[pl.ds(3 * T * E, E)]],
                      psb[p], True))
        descs.append((seq_table.at[idxb[p].at[pl.ds(3 * T * E + E, E)]],
                      psb[p], True))
        return descs

    def fire(p):
        for src, dst, add in g_list(p):
            pltpu.async_copy(src, dst, sem_g[p], add=add)

    def wait(p):
        for src, dst, _ in g_list(p):
            pltpu.make_async_copy(src, dst, sem_g[p]).wait()

    def compute_and_store(p, c):
        ilane = lax.iota(jnp.int32, LANES)

        def event_body(e, _):
            wv = wb[p][pl.ds(e * T, LANES)]
            ev = jnp.zeros((LANES,), jnp.int32) + e
            for dc in range(D // LANES):
                sl = pl.ds(dc * LANES, LANES)
                acc = psb[p][e, sl]
                for t in range(T):
                    acc = acc + rowsb[p][t, e, sl] * wv[t]
                plsc.store_scatter(outt, [ilane + dc * LANES, ev], acc)
            return 0
        lax.fori_loop(0, E, event_body, 0)

        g = g0 + c
        sl_ix = lax.shift_right_logical(g, shift_nb)
        bb = lax.bitwise_and(g, nb - 1)
        pltpu.sync_copy(outt, out_hbm.at[sl_ix, :, pl.ds(bb * E, E)])

    pltpu.sync_copy(ids_packed.at[g0], idx0)
    _weights(idx0, w0)
    zero_rows(0)
    fire(0)
    pltpu.make_async_copy(ids_packed.at[g0 + 1], idx1, sem_idx).start()

    def step(i, _):
        for p in (0, 1):
            c = i * 2 + p
            q = 1 - p

            @pl.when(c + 1 < nc)
            def _prefetch():
                pltpu.make_async_copy(
                    ids_packed.at[g0 + c + 1], idxb[q], sem_idx).wait()
                _weights(idxb[q], wb[q])
                zero_rows(q)
                fire(q)

            wait(p)

            @pl.when(c + 2 < nc)
            def _next_idx():
                pltpu.make_async_copy(
                    ids_packed.at[g0 + c + 2], idxb[p], sem_idx).start()

            compute_and_store(p, c)
        return 0

    lax.fori_loop(0, nc // 2, step, 0)


def _k2_body(nc, nb, ids_packed, token_table, part_hbm, out_hbm,
             idx0, idx1, w0, w1, rows0, rows1, outt,
             sem_idx, sem_g0, sem_g1):
    idxb, wb = [idx0, idx1], [w0, w1]
    rowsb = [rows0, rows1]
    sem_g = [sem_g0, sem_g1]
    shift_nb, g0 = _common(nc, nb)

    def g_list(p):
        return [(token_table.at[idxb[p].at[pl.ds(t * E, E)]], rowsb[p].at[t])
                for t in range(T)]

    def fire(p):
        for src, dst in g_list(p):
            pltpu.async_copy(src, dst, sem_g[p])

    def wait(p):
        for src, dst in g_list(p):
            pltpu.make_async_copy(src, dst, sem_g[p]).wait()

    def idx_copy(c, p, sem):
        return pltpu.make_async_copy(
            ids_packed.at[g0 + c, pl.ds(0, T * E)], idxb[p], sem)

    def compute_and_store(p, c):
        ilane = lax.iota(jnp.int32, LANES)
        g = g0 + c
        sl_ix = lax.shift_right_logical(g, shift_nb)
        bb = lax.bitwise_and(g, nb - 1)
        pltpu.sync_copy(part_hbm.at[sl_ix, :, pl.ds(bb * E, E)], outt)

        def event_body(e, _):
            wv = wb[p][pl.ds(e * T, LANES)]
            ev = jnp.zeros((LANES,), jnp.int32) + e
            for dc in range(D // LANES):
                sl = pl.ds(dc * LANES, LANES)
                acc = rowsb[p][0, e, sl] * wv[0]
                for t in range(1, T):
                    acc = acc + rowsb[p][t, e, sl] * wv[t]
                plsc.addupdate_scatter(outt, [ilane + dc * LANES, ev], acc)
            return 0
        lax.fori_loop(0, E, event_body, 0)

        pltpu.sync_copy(outt, out_hbm.at[sl_ix, :, pl.ds(bb * E, E)])

    idx_copy(0, 0, sem_idx).start()
    idx_copy(0, 0, sem_idx).wait()
    _weights(idx0, w0)
    fire(0)
    idx_copy(1, 1, sem_idx).start()

    def step(i, _):
        for p in (0, 1):
            c = i * 2 + p
            q = 1 - p

            @pl.when(c + 1 < nc)
            def _prefetch():
                idx_copy(c + 1, q, sem_idx).wait()
                _weights(idxb[q], wb[q])
                fire(q)

            wait(p)

            @pl.when(c + 2 < nc)
            def _next_idx():
                idx_copy(c + 2, p, sem_idx).start()

            compute_and_store(p, c)
        return 0

    lax.fori_loop(0, nc // 2, step, 0)


@functools.partial(jax.jit, static_argnums=())
def kernel(token_ids, feature_ids, type_ids, pos_ids, seq_name_ids,
           token_table, feature_table, type_table, seq_table, pos_table):
    B, S, L, Tt = token_ids.shape
    assert Tt == T and token_table.shape[1] == D
    SL = S * L
    nb = B // E
    NCHUNKS = SL * nb
    assert NCHUNKS % W == 0
    nc = NCHUNKS // W
    assert nc % 2 == 0

    def plane3(x):
        return (x.astype(jnp.int32).transpose(1, 2, 3, 0)
                .reshape(SL, T, nb, E).transpose(0, 2, 1, 3))

    def plane1(x):
        return (x.astype(jnp.int32).transpose(1, 2, 0)
                .reshape(SL, nb, 1, E))

    ids_packed = jnp.concatenate(
        [plane3(token_ids), plane3(feature_ids), plane3(type_ids),
         plane1(pos_ids), plane1(seq_name_ids)],
        axis=2).reshape(NCHUNKS, PLANE)

    mesh = plsc.VectorSubcoreMesh(core_axis_name="c", subcore_axis_name="s",
                                  num_cores=2, num_subcores=16)
    cparams = pltpu.CompilerParams(use_tc_tiling_on_sc=False,
                                   needs_layout_passes=False)

    k1 = pl.kernel(
        functools.partial(_k1_body, nc, nb),
        out_type=jax.ShapeDtypeStruct((SL, D, B), jnp.float32),
        mesh=mesh,
        compiler_params=cparams,
        scratch_types=[
            pltpu.VMEM((PLANE,), jnp.int32),
            pltpu.VMEM((PLANE,), jnp.int32),
            pltpu.VMEM((T * E + LANES,), jnp.float32),
            pltpu.VMEM((T * E + LANES,), jnp.float32),
            pltpu.VMEM((T, E, D), jnp.float32),
            pltpu.VMEM((T, E, D), jnp.float32),
            pltpu.VMEM((E, D), jnp.float32),
            pltpu.VMEM((E, D), jnp.float32),
            pltpu.VMEM((D, E), jnp.float32),
            pltpu.SemaphoreType.DMA,
            pltpu.SemaphoreType.DMA,
            pltpu.SemaphoreType.DMA,
        ],
    )
    part = k1(ids_packed, feature_table, type_table, seq_table, pos_table)

    k2 = pl.kernel(
        functools.partial(_k2_body, nc, nb),
        out_type=jax.ShapeDtypeStruct((SL, D, B), jnp.float32),
        mesh=mesh,
        compiler_params=cparams,
        scratch_types=[
            pltpu.VMEM((T * E,), jnp.int32),
            pltpu.VMEM((T * E,), jnp.int32),
            pltpu.VMEM((T * E + LANES,), jnp.float32),
            pltpu.VMEM((T * E + LANES,), jnp.float32),
            pltpu.VMEM((T, E, D), jnp.float32),
            pltpu.VMEM((T, E, D), jnp.float32),
            pltpu.VMEM((D, E), jnp.float32),
            pltpu.SemaphoreType.DMA,
            pltpu.SemaphoreType.DMA,
            pltpu.SemaphoreType.DMA,
        ],
    )
    out = k2(ids_packed, token_table, part)

    return out.reshape(S, L, D, B).transpose(3, 0, 1, 2)


# speedup vs baseline: 1.3533x; 1.1829x over previous
import functools

import jax
import jax.numpy as jnp
from jax import lax
from jax.experimental import pallas as pl
from jax.experimental.pallas import tpu as pltpu
from jax.experimental.pallas import tpu_sc as plsc

D = 64
T = 4
E = 128
LANES = 16
W = 32
PLANE = 14 * E
HIMASK = -65536


def _sc_kernel_body(nc, nb, ids_packed,
                    token_table, feature_table, type_table, seq_table,
                    pos_table, out_hbm,
                    idx0, idx1, w0, w1, rows0, rows1, ps0, ps1, outt,
                    sem_idx, sem_g0, sem_g1):
    idxb = [idx0, idx1]
    wb = [w0, w1]
    rowsb = [rows0, rows1]
    psb = [ps0, ps1]
    sem_g = [sem_g0, sem_g1]

    assert nb & (nb - 1) == 0
    shift_nb = nb.bit_length() - 1
    wid = lax.axis_index("s") * 2 + lax.axis_index("c")
    g0 = wid * nc

    def zero_rows(p):
        z16 = jnp.zeros((LANES,), jnp.float32)

        def zr(e, _):
            for dc in range(D // LANES):
                sl = pl.ds(dc * LANES, LANES)
                for t in range(T):
                    rowsb[p][t, e, sl] = z16
                psb[p][e, sl] = z16
            return 0
        lax.fori_loop(0, E, zr, 0)

    def fire_gathers(p):
        for k, table in enumerate((token_table, feature_table, type_table)):
            for t in range(T):
                pltpu.async_copy(
                    table.at[idxb[p].at[pl.ds((k * T + t) * E, E)]],
                    rowsb[p].at[t], sem_g[p], add=True)
        pltpu.async_copy(
            pos_table.at[idxb[p].at[pl.ds(3 * T * E, E)]],
            psb[p], sem_g[p], add=True)
        pltpu.async_copy(
            seq_table.at[idxb[p].at[pl.ds(3 * T * E + E, E)]],
            psb[p], sem_g[p], add=True)

    def wait_gathers(p):
        for k, table in enumerate((token_table, feature_table, type_table)):
            for t in range(T):
                pltpu.make_async_copy(
                    table.at[idxb[p].at[pl.ds((k * T + t) * E, E)]],
                    rowsb[p].at[t], sem_g[p]).wait()
        pltpu.make_async_copy(
            pos_table.at[idxb[p].at[pl.ds(3 * T * E, E)]],
            psb[p], sem_g[p]).wait()
        pltpu.make_async_copy(
            seq_table.at[idxb[p].at[pl.ds(3 * T * E + E, E)]],
            psb[p], sem_g[p]).wait()

    def weights(p):
        ilane = lax.iota(jnp.int32, LANES)

        def grp(i, _):
            cnt = jnp.zeros((LANES,), jnp.float32)
            ms = []
            for t in range(T):
                ids = idxb[p][pl.ds(t * E + i * LANES, LANES)]
                m = jnp.where(ids != 0, 1.0, 0.0).astype(jnp.float32)
                ms.append(m)
                cnt = cnt + m
            rv = 1.0 / jnp.maximum(cnt, 1.0)
            widx = (ilane + i * LANES) * T
            for t in range(T):
                plsc.store_scatter(wb[p], [widx + t], ms[t] * rv)
            return 0
        lax.fori_loop(0, E // LANES, grp, 0)

    def compute_and_store(p, c):
        ilane = lax.iota(jnp.int32, LANES)

        def event_body(e, _):
            wv = wb[p][pl.ds(e * T, LANES)]
            ev = jnp.zeros((LANES,), jnp.int32) + e
            for dc in range(D // LANES):
                sl = pl.ds(dc * LANES, LANES)
                acc = psb[p][e, sl]
                for t in range(T):
                    acc = acc + rowsb[p][t, e, sl] * wv[t]
                plsc.store_scatter(outt, [ilane + dc * LANES, ev], acc)
            return 0
        lax.fori_loop(0, E, event_body, 0)

        g = g0 + c
        sl_ix = lax.shift_right_logical(g, shift_nb)
        bb = lax.bitwise_and(g, nb - 1)
        pltpu.sync_copy(
            outt, out_hbm.at[sl_ix, :, pl.ds(bb * E, E)])

    pltpu.sync_copy(ids_packed.at[g0], idx0)
    weights(0)
    zero_rows(0)
    fire_gathers(0)
    pltpu.make_async_copy(ids_packed.at[g0 + 1], idx1, sem_idx).start()

    def step(i, _):
        for p in (0, 1):
            c = i * 2 + p
            q = 1 - p

            @pl.when(c + 1 < nc)
            def _prefetch():
                pltpu.make_async_copy(
                    ids_packed.at[g0 + c + 1], idxb[q], sem_idx).wait()
                weights(q)
                zero_rows(q)
                fire_gathers(q)

            wait_gathers(p)

            @pl.when(c + 2 < nc)
            def _next_idx():
                pltpu.make_async_copy(
                    ids_packed.at[g0 + c + 2], idxb[p], sem_idx).start()

            compute_and_store(p, c)
        return 0

    lax.fori_loop(0, nc // 2, step, 0)


@functools.partial(jax.jit, static_argnums=())
def kernel(token_ids, feature_ids, type_ids, pos_ids, seq_name_ids,
           token_table, feature_table, type_table, seq_table, pos_table):
    B, S, L, Tt = token_ids.shape
    assert Tt == T and token_table.shape[1] == D
    SL = S * L
    N = B * SL
    nb = B // E
    NCHUNKS = SL * nb
    assert NCHUNKS % W == 0
    nc = NCHUNKS // W
    assert nc % 2 == 0

    def plane3(x):
        return (x.astype(jnp.int32).transpose(1, 2, 3, 0)
                .reshape(SL, T, nb, E).transpose(0, 2, 1, 3))

    def plane1(x):
        return (x.astype(jnp.int32).transpose(1, 2, 0)
                .reshape(SL, nb, 1, E))

    ids_packed = jnp.concatenate(
        [plane3(token_ids), plane3(feature_ids), plane3(type_ids),
         plane1(pos_ids), plane1(seq_name_ids)],
        axis=2).reshape(NCHUNKS, PLANE)

    mesh = plsc.VectorSubcoreMesh(core_axis_name="c", subcore_axis_name="s",
                                  num_cores=2, num_subcores=16)
    kfn = pl.kernel(
        functools.partial(_sc_kernel_body, nc, nb),
        out_type=jax.ShapeDtypeStruct((SL, D, B), jnp.float32),
        mesh=mesh,
        compiler_params=pltpu.CompilerParams(use_tc_tiling_on_sc=False,
                                             needs_layout_passes=False),
        scratch_types=[
            pltpu.VMEM((PLANE,), jnp.int32),
            pltpu.VMEM((PLANE,), jnp.int32),
            pltpu.VMEM((T * E + LANES,), jnp.float32),
            pltpu.VMEM((T * E + LANES,), jnp.float32),
            pltpu.VMEM((T, E, D), jnp.float32),
            pltpu.VMEM((T, E, D), jnp.float32),
            pltpu.VMEM((E, D), jnp.float32),
            pltpu.VMEM((E, D), jnp.float32),
            pltpu.VMEM((D, E), jnp.float32),
            pltpu.SemaphoreType.DMA,
            pltpu.SemaphoreType.DMA,
            pltpu.SemaphoreType.DMA,
        ],
    )
    out = kfn(ids_packed, token_table, feature_table, type_table,
              seq_table, pos_table)
    return out.reshape(S, L, D, B).transpose(3, 0, 1, 2)
